# baseline (speedup 1.0000x reference)
import functools

import jax
import jax.numpy as jnp
from jax import lax
from jax.experimental import pallas as pl
from jax.experimental.pallas import tpu as pltpu
from jax.experimental.pallas import tpu_sc as plsc

NC = 2
NS = 16
NW = NC * NS


def _make_sc_gather(V, D, sizes):
    bs = [B // NW for B in sizes]
    ch = [min(b, 128) for b in bs]
    nc = [b // c for b, c in zip(bs, ch)]
    mesh = plsc.VectorSubcoreMesh(core_axis_name="c", subcore_axis_name="s")
    scratch = []
    for b, c, n in zip(bs, ch, nc):
        scratch += [pltpu.VMEM((n, c), jnp.int32),
                    pltpu.VMEM((b, D), jnp.float32)]
    scratch += [pltpu.SemaphoreType.DMA, pltpu.SemaphoreType.DMA]
    G = len(sizes)

    @functools.partial(
        pl.kernel,
        out_type=tuple(jax.ShapeDtypeStruct((B, D), jnp.float32)
                       for B in sizes),
        mesh=mesh,
        scratch_types=scratch,
    )
    def gather(table, *refs):
        idxs = refs[:G]
        outs = refs[G:2 * G]
        scr = refs[2 * G:]
        gsem, wsem = scr[-2], scr[-1]
        wid = lax.axis_index("s") * NC + lax.axis_index("c")
        fired = []
        for g in range(G):
            iv, rv = scr[2 * g], scr[2 * g + 1]
            base = wid * bs[g]
            for j in range(nc[g]):
                pltpu.sync_copy(idxs[g].at[pl.ds(base + j * ch[g], ch[g])],
                                iv.at[j])
            for j in range(nc[g]):
                fired.append((pltpu.async_copy(
                    table.at[iv.at[j]],
                    rv.at[pl.ds(j * ch[g], ch[g])], gsem), g, j))
        wbs = []
        for cp, g, j in fired:
            cp.wait()
            rv = scr[2 * g + 1]
            base = wid * bs[g]
            wbs.append(pltpu.async_copy(
                rv.at[pl.ds(j * ch[g], ch[g])],
                outs[g].at[pl.ds(base + j * ch[g], ch[g])], wsem))
        for cp in wbs:
            cp.wait()

    return gather


def _mm_part(dif, g, kblk_off, p, tail, bk):
    M = dif.shape[0]
    Ks, D = g.shape
    nk = Ks // bk

    dif_spec = pl.BlockSpec((M, bk), lambda k: (0, k + kblk_off))
    g_spec = pl.BlockSpec((bk, D), lambda k: (k, 0))
    full = pl.BlockSpec((M, D), lambda k: (0, 0))
    wspec = pl.BlockSpec((D, D), lambda k: (0, 0))

    if tail is None:
        def body(dif_ref, g_ref, *rest):
            (p_ref, out_ref) = ((rest[0], rest[1]) if p is not None
                                else (None, rest[0]))
            k = pl.program_id(0)
            contrib = jnp.dot(dif_ref[...], g_ref[...],
                              preferred_element_type=jnp.float32)

            @pl.when(k == 0)
            def _():
                out_ref[...] = (contrib if p_ref is None
                                else p_ref[...] + contrib)

            @pl.when(k > 0)
            def _():
                out_ref[...] += contrib

        in_specs = [dif_spec, g_spec] + ([full] if p is not None else [])
        args = (dif, g) + ((p,) if p is not None else ())
        return pl.pallas_call(
            body,
            grid=(nk,),
            in_specs=in_specs,
            out_specs=full,
            out_shape=jax.ShapeDtypeStruct((M, D), jnp.float32),
        )(*args)

    d2, wa, wb = tail

    def body(dif_ref, g_ref, *rest):
        if p is not None:
            p_ref, d2_ref, wa_ref, wb_ref, out_ref, acc_ref = rest
        else:
            d2_ref, wa_ref, wb_ref, out_ref, acc_ref = rest
            p_ref = None
        k = pl.program_id(0)
        contrib = jnp.dot(dif_ref[...], g_ref[...],
                          preferred_element_type=jnp.float32)

        @pl.when(k == 0)
        def _():
            acc_ref[...] = (contrib if p_ref is None
                            else p_ref[...] + contrib)

        @pl.when(k > 0)
        def _():
            acc_ref[...] += contrib

        @pl.when(k == nk - 1)
        def _():
            out_ref[...] = jnp.maximum(
                jnp.dot(acc_ref[...], wa_ref[...],
                        preferred_element_type=jnp.float32)
                + jnp.dot(d2_ref[...], wb_ref[...],
                          preferred_element_type=jnp.float32),
                0.0)

    in_specs = ([dif_spec, g_spec]
                + ([full] if p is not None else [])
                + [full, wspec, wspec])
    args = (dif, g) + ((p,) if p is not None else ()) + (d2, wa, wb)
    return pl.pallas_call(
        body,
        grid=(nk,),
        in_specs=in_specs,
        out_specs=full,
        out_shape=jax.ShapeDtypeStruct((M, D), jnp.float32),
        scratch_shapes=[pltpu.VMEM((M, D), jnp.float32)],
    )(*args)


def _mm_small(dif, g, d1, wa, wb):
    M = dif.shape[0]
    D = g.shape[1]

    def body(dif_ref, g_ref, d_ref, wa_ref, wb_ref, out_ref):
        agg = jnp.dot(dif_ref[...], g_ref[...],
                      preferred_element_type=jnp.float32)
        out_ref[...] = jnp.maximum(
            jnp.dot(agg, wa_ref[...], preferred_element_type=jnp.float32)
            + jnp.dot(d_ref[...], wb_ref[...],
                      preferred_element_type=jnp.float32),
            0.0)

    return pl.pallas_call(
        body,
        out_shape=jax.ShapeDtypeStruct((M, D), jnp.float32),
    )(dif, g, d1, wa, wb)


K_HEAD = 4096
BK_HEAD = 1024
BK_TAIL = 2048


def kernel(src_nodes, dstsrc2src_1, dstsrc2src_2, dstsrc2dst_1, dstsrc2dst_2,
           dif_mat_1, dif_mat_2, w1, w2):
    V, D = src_nodes.shape
    w1a, w1b = w1[:D], w1[D:]
    w2a, w2b = w2[:D], w2[D:]
    K = dstsrc2src_2.shape[0]
    K_tail = K - K_HEAD

    g_head = _make_sc_gather(V, D, (K_HEAD,))
    (g0,) = g_head(src_nodes, dstsrc2src_2[:K_HEAD])
    g_tail = _make_sc_gather(V, D, (K_tail, dstsrc2dst_2.shape[0]))
    g1, d2 = g_tail(src_nodes, dstsrc2src_2[K_HEAD:], dstsrc2dst_2)

    p = _mm_part(dif_mat_2, g0, 0, None, None, BK_HEAD)
    x = _mm_part(dif_mat_2, g1, K_HEAD // BK_TAIL, p, (d2, w1a, w1b), BK_TAIL)

    gather2 = _make_sc_gather(x.shape[0], D,
                              (dstsrc2src_1.shape[0], dstsrc2dst_1.shape[0]))
    g1, d1 = gather2(x, dstsrc2src_1, dstsrc2dst_1)
    return _mm_small(dif_mat_1, g1, d1, w2a, w2b)


# speedup vs baseline: 1.0460x; 1.0460x over previous
import functools

import jax
import jax.numpy as jnp
from jax import lax
from jax.experimental import pallas as pl
from jax.experimental.pallas import tpu as pltpu
from jax.experimental.pallas import tpu_sc as plsc

NC = 2
NS = 16
NW = NC * NS


def _make_sc_gather(V, D, sizes):
    bs = [B // NW for B in sizes]
    ch = [min(b, 128) for b in bs]
    nc = [b // c for b, c in zip(bs, ch)]
    mesh = plsc.VectorSubcoreMesh(core_axis_name="c", subcore_axis_name="s")
    scratch = []
    for b, c, n in zip(bs, ch, nc):
        scratch += [pltpu.VMEM((n, c), jnp.int32),
                    pltpu.VMEM((b, D), jnp.float32)]
    scratch += [pltpu.SemaphoreType.DMA, pltpu.SemaphoreType.DMA]
    G = len(sizes)

    @functools.partial(
        pl.kernel,
        out_type=tuple(jax.ShapeDtypeStruct((B, D), jnp.float32)
                       for B in sizes),
        mesh=mesh,
        scratch_types=scratch,
    )
    def gather(table, *refs):
        idxs = refs[:G]
        outs = refs[G:2 * G]
        scr = refs[2 * G:]
        gsem, wsem = scr[-2], scr[-1]
        wid = lax.axis_index("s") * NC + lax.axis_index("c")
        fired = []
        for g in range(G):
            iv, rv = scr[2 * g], scr[2 * g + 1]
            base = wid * bs[g]
            for j in range(nc[g]):
                pltpu.sync_copy(idxs[g].at[pl.ds(base + j * ch[g], ch[g])],
                                iv.at[j])
            for j in range(nc[g]):
                fired.append((pltpu.async_copy(
                    table.at[iv.at[j]],
                    rv.at[pl.ds(j * ch[g], ch[g])], gsem), g, j))
        wbs = []
        for cp, g, j in fired:
            cp.wait()
            rv = scr[2 * g + 1]
            base = wid * bs[g]
            wbs.append(pltpu.async_copy(
                rv.at[pl.ds(j * ch[g], ch[g])],
                outs[g].at[pl.ds(base + j * ch[g], ch[g])], wsem))
        for cp in wbs:
            cp.wait()

    return gather


def _mm_part(dif, g, kblk_off, p, tail, bk):
    M = dif.shape[0]
    Ks, D = g.shape
    nk = Ks // bk

    dif_spec = pl.BlockSpec((M, bk), lambda k: (0, k + kblk_off))
    g_spec = pl.BlockSpec((bk, D), lambda k: (k, 0))
    full = pl.BlockSpec((M, D), lambda k: (0, 0))
    wspec = pl.BlockSpec((D, D), lambda k: (0, 0))

    if tail is None:
        def body(dif_ref, g_ref, *rest):
            (p_ref, out_ref) = ((rest[0], rest[1]) if p is not None
                                else (None, rest[0]))
            k = pl.program_id(0)
            contrib = jnp.dot(dif_ref[...], g_ref[...],
                              preferred_element_type=jnp.float32)

            @pl.when(k == 0)
            def _():
                out_ref[...] = (contrib if p_ref is None
                                else p_ref[...] + contrib)

            @pl.when(k > 0)
            def _():
                out_ref[...] += contrib

        in_specs = [dif_spec, g_spec] + ([full] if p is not None else [])
        args = (dif, g) + ((p,) if p is not None else ())
        return pl.pallas_call(
            body,
            grid=(nk,),
            in_specs=in_specs,
            out_specs=full,
            out_shape=jax.ShapeDtypeStruct((M, D), jnp.float32),
        )(*args)

    d2, wa, wb = tail

    def body(dif_ref, g_ref, *rest):
        if p is not None:
            p_ref, d2_ref, wa_ref, wb_ref, out_ref, acc_ref = rest
        else:
            d2_ref, wa_ref, wb_ref, out_ref, acc_ref = rest
            p_ref = None
        k = pl.program_id(0)
        contrib = jnp.dot(dif_ref[...], g_ref[...],
                          preferred_element_type=jnp.float32)

        @pl.when(k == 0)
        def _():
            acc_ref[...] = (contrib if p_ref is None
                            else p_ref[...] + contrib)

        @pl.when(k > 0)
        def _():
            acc_ref[...] += contrib

        @pl.when(k == nk - 1)
        def _():
            out_ref[...] = jnp.maximum(
                jnp.dot(acc_ref[...], wa_ref[...],
                        preferred_element_type=jnp.float32)
                + jnp.dot(d2_ref[...], wb_ref[...],
                          preferred_element_type=jnp.float32),
                0.0)

    in_specs = ([dif_spec, g_spec]
                + ([full] if p is not None else [])
                + [full, wspec, wspec])
    args = (dif, g) + ((p,) if p is not None else ()) + (d2, wa, wb)
    return pl.pallas_call(
        body,
        grid=(nk,),
        in_specs=in_specs,
        out_specs=full,
        out_shape=jax.ShapeDtypeStruct((M, D), jnp.float32),
        scratch_shapes=[pltpu.VMEM((M, D), jnp.float32)],
    )(*args)


def _mm_small_onehot(dif, x, idx_s, idx_d, wa, wb):
    M, Kv = dif.shape
    V2, D = x.shape

    def body(dif_ref, x_ref, is_ref, id_ref, wa_ref, wb_ref, out_ref):
        ids_s = jax.lax.broadcasted_iota(jnp.int32, (Kv, V2), 1)
        o_s = (ids_s == is_ref[...]).astype(jnp.float32)
        g1 = jnp.dot(o_s, x_ref[...], preferred_element_type=jnp.float32)
        ids_d = jax.lax.broadcasted_iota(jnp.int32, (M, V2), 1)
        o_d = (ids_d == id_ref[...]).astype(jnp.float32)
        d1 = jnp.dot(o_d, x_ref[...], preferred_element_type=jnp.float32)
        agg = jnp.dot(dif_ref[...], g1, preferred_element_type=jnp.float32)
        out_ref[...] = jnp.maximum(
            jnp.dot(agg, wa_ref[...], preferred_element_type=jnp.float32)
            + jnp.dot(d1, wb_ref[...], preferred_element_type=jnp.float32),
            0.0)

    return pl.pallas_call(
        body,
        out_shape=jax.ShapeDtypeStruct((M, D), jnp.float32),
    )(dif, x, idx_s.reshape(Kv, 1), idx_d.reshape(M, 1), wa, wb)


BK = 1024


def kernel(src_nodes, dstsrc2src_1, dstsrc2src_2, dstsrc2dst_1, dstsrc2dst_2,
           dif_mat_1, dif_mat_2, w1, w2):
    V, D = src_nodes.shape
    w1a, w1b = w1[:D], w1[D:]
    w2a, w2b = w2[:D], w2[D:]

    gather1 = _make_sc_gather(V, D,
                              (dstsrc2src_2.shape[0], dstsrc2dst_2.shape[0]))
    g2, d2 = gather1(src_nodes, dstsrc2src_2, dstsrc2dst_2)
    x = _mm_part(dif_mat_2, g2, 0, None, (d2, w1a, w1b), BK)

    return _mm_small_onehot(dif_mat_1, x, dstsrc2src_1, dstsrc2dst_1,
                            w2a, w2b)
